# baseline (speedup 1.0000x reference)
import functools
import numpy as np
import jax
import jax.numpy as jnp
from jax import lax
from jax.experimental import pallas as pl
from jax.experimental.pallas import tpu as pltpu
from jax.experimental.pallas import tpu_sc as plsc

_DIFFUSION_STEPS = 1000
_BETA_START = 0.0001
_BETA_END = 0.02


def _make_tables():
    betas = np.linspace(_BETA_START, _BETA_END, _DIFFUSION_STEPS, dtype=np.float32)
    alphas = (np.float32(1.0) - betas).astype(np.float32)
    alphas_cumprod = np.cumprod(alphas, dtype=np.float32)
    sqrt_acp = np.sqrt(alphas_cumprod).astype(np.float32)
    sqrt_omacp = np.sqrt((np.float32(1.0) - alphas_cumprod)).astype(np.float32)
    return sqrt_acp, sqrt_omacp


_SQRT_ACP, _SQRT_OMACP = _make_tables()

_NC = 2
_NS = 16
_NW = _NC * _NS
_L = 16

_ROWS_PER_CHUNK = 64


def _sc_body(ts_hbm, taba_hbm, tabb_hbm, x_hbm, n_hbm, out_hbm,
             ts_v, taba_v, tabb_v, xb0, xb1, nb0, nb1, ob0, ob1,
             sx0, sx1, sn0, sn1, so0, so1):
    wid = lax.axis_index("s") * _NC + lax.axis_index("c")

    pltpu.sync_copy(ts_hbm, ts_v)
    pltpu.sync_copy(taba_hbm, taba_v)
    pltpu.sync_copy(tabb_hbm, tabb_v)

    widx = jnp.full((_L,), wid, dtype=jnp.int32)
    tsvec = plsc.load_gather(ts_v, [widx])
    avec = plsc.load_gather(taba_v, [tsvec])
    bvec = plsc.load_gather(tabb_v, [tsvec])

    ch, h, w = x_hbm.shape[1], x_hbm.shape[2], x_hbm.shape[3]
    rpc = _ROWS_PER_CHUNK
    chunks = []
    for c in range(ch):
        for r0 in range(0, h, rpc):
            chunks.append((c, r0))
    n_chunks = len(chunks)

    xbufs = (xb0, xb1)
    nbufs = (nb0, nb1)
    obufs = (ob0, ob1)
    sxs = (sx0, sx1)
    sns = (sn0, sn1)
    sos = (so0, so1)

    def start_in(ci):
        c, r0 = chunks[ci]
        s = ci % 2
        hx = pltpu.async_copy(x_hbm.at[wid, c, pl.ds(r0, rpc), :], xbufs[s], sxs[s])
        hn = pltpu.async_copy(n_hbm.at[wid, c, pl.ds(r0, rpc), :], nbufs[s], sns[s])
        return hx, hn

    def compute(s):
        xv, nv, ov = xbufs[s], nbufs[s], obufs[s]

        def body(r, _):
            for l in range(0, w, _L):
                ov[r, pl.ds(l, _L)] = avec * xv[r, pl.ds(l, _L)] + bvec * nv[r, pl.ds(l, _L)]
            return 0

        lax.fori_loop(0, rpc, body, 0)

    def start_out(ci):
        c, r0 = chunks[ci]
        s = ci % 2
        return pltpu.async_copy(obufs[s], out_hbm.at[wid, c, pl.ds(r0, rpc), :], sos[s])

    in_handles = [None] * n_chunks
    out_handles = [None] * n_chunks

    in_handles[0] = start_in(0)
    for ci in range(n_chunks):
        s = ci % 2
        if ci + 1 < n_chunks:
            in_handles[ci + 1] = start_in(ci + 1)
        hx, hn = in_handles[ci]
        hx.wait()
        hn.wait()
        if ci >= 2:
            out_handles[ci - 2].wait()
        compute(s)
        out_handles[ci] = start_out(ci)
    out_handles[n_chunks - 2].wait()
    out_handles[n_chunks - 1].wait()


def kernel(clean_future, timesteps, noise):
    batch, ch, h, w = clean_future.shape

    mesh = plsc.VectorSubcoreMesh(core_axis_name="c", subcore_axis_name="s")
    rpc = _ROWS_PER_CHUNK

    sc_call = functools.partial(
        pl.kernel,
        mesh=mesh,
        out_type=jax.ShapeDtypeStruct(clean_future.shape, jnp.float32),
        scratch_types=[
            pltpu.VMEM((batch,), jnp.int32),
            pltpu.VMEM((_DIFFUSION_STEPS,), jnp.float32),
            pltpu.VMEM((_DIFFUSION_STEPS,), jnp.float32),
            pltpu.VMEM((rpc, w), jnp.float32),
            pltpu.VMEM((rpc, w), jnp.float32),
            pltpu.VMEM((rpc, w), jnp.float32),
            pltpu.VMEM((rpc, w), jnp.float32),
            pltpu.VMEM((rpc, w), jnp.float32),
            pltpu.VMEM((rpc, w), jnp.float32),
            pltpu.SemaphoreType.DMA,
            pltpu.SemaphoreType.DMA,
            pltpu.SemaphoreType.DMA,
            pltpu.SemaphoreType.DMA,
            pltpu.SemaphoreType.DMA,
            pltpu.SemaphoreType.DMA,
        ],
        compiler_params=pltpu.CompilerParams(needs_layout_passes=False),
    )(_sc_body)

    out = sc_call(
        timesteps.astype(jnp.int32),
        jnp.asarray(_SQRT_ACP),
        jnp.asarray(_SQRT_OMACP),
        clean_future,
        noise,
    )
    return out, noise


# speedup vs baseline: 2.1534x; 2.1534x over previous
import numpy as np
import jax
import jax.numpy as jnp
from jax.experimental import pallas as pl
from jax.experimental.pallas import tpu as pltpu

_DIFFUSION_STEPS = 1000
_BETA_START = 0.0001
_BETA_END = 0.02


def _make_tables():
    betas = np.linspace(_BETA_START, _BETA_END, _DIFFUSION_STEPS, dtype=np.float32)
    alphas = (np.float32(1.0) - betas).astype(np.float32)
    alphas_cumprod = np.cumprod(alphas, dtype=np.float32)
    sqrt_acp = np.sqrt(alphas_cumprod).astype(np.float32)
    sqrt_omacp = np.sqrt((np.float32(1.0) - alphas_cumprod)).astype(np.float32)
    return sqrt_acp, sqrt_omacp


_SQRT_ACP, _SQRT_OMACP = _make_tables()

_LANES = 128


_SAMPLES_PER_BLOCK = 4


def _noise_body(ts_ref, a_tab_ref, b_tab_ref, x_ref, n_ref, o_ref, n_out_ref):
    i = pl.program_id(0)
    for s in range(_SAMPLES_PER_BLOCK):
        t = ts_ref[i * _SAMPLES_PER_BLOCK + s]
        a = a_tab_ref[t]
        b = b_tab_ref[t]
        nv = n_ref[s]
        o_ref[s] = a * x_ref[s] + b * nv
        n_out_ref[s] = nv


def kernel(clean_future, timesteps, noise):
    batch, ch, h, w = clean_future.shape

    spb = _SAMPLES_PER_BLOCK
    block = (spb, ch, h, w)
    grid_spec = pltpu.PrefetchScalarGridSpec(
        num_scalar_prefetch=3,
        grid=(batch // spb,),
        in_specs=[
            pl.BlockSpec(block, lambda i, *_: (i, 0, 0, 0)),
            pl.BlockSpec(block, lambda i, *_: (i, 0, 0, 0)),
        ],
        out_specs=[
            pl.BlockSpec(block, lambda i, *_: (i, 0, 0, 0)),
            pl.BlockSpec(block, lambda i, *_: (i, 0, 0, 0)),
        ],
    )

    out, n_out = pl.pallas_call(
        _noise_body,
        grid_spec=grid_spec,
        out_shape=[
            jax.ShapeDtypeStruct(clean_future.shape, jnp.float32),
            jax.ShapeDtypeStruct(clean_future.shape, jnp.float32),
        ],
    )(timesteps, jnp.asarray(_SQRT_ACP), jnp.asarray(_SQRT_OMACP), clean_future, noise)

    return out, n_out
